# baseline (speedup 1.0000x reference)
import jax
import jax.numpy as jnp
from jax import lax
from jax.experimental import pallas as pl
from jax.experimental.pallas import tpu as pltpu
from jax.experimental.pallas import tpu_sc as plsc

VOCAB = 1_000_000
D = 64
BATCH = 4096
SEQ = 50
PW = 128

_info = plsc.get_sparse_core_info()
NC = _info.num_cores
NS = _info.num_subcores
NW = NC * NS
CH = BATCH // NW
NB = 2
L = 16


def _make_lookup():
  mesh = plsc.VectorSubcoreMesh(core_axis_name="c", subcore_axis_name="s")

  @pl.kernel(
      out_type=jax.ShapeDtypeStruct((SEQ, D, BATCH), jnp.float32),
      mesh=mesh,
      compiler_params=pltpu.CompilerParams(needs_layout_passes=False),
      scratch_types=(
          [pltpu.VMEM((SEQ, CH), jnp.int32),
           pltpu.VMEM((SEQ, CH), jnp.int32)]
          + [pltpu.VMEM((CH, PW), jnp.float32) for _ in range(NB)]
          + [pltpu.VMEM((D, CH), jnp.float32) for _ in range(NB)]
          + [pltpu.SemaphoreType.DMA for _ in range(2 * NB)]
      ),
  )
  def lookup(tp_hbm, xt_hbm, out_hbm, idx_v, par_v, *bufs_sems):
    gbufs = bufs_sems[:NB]
    tbufs = bufs_sems[NB:2 * NB]
    sg = bufs_sems[2 * NB:3 * NB]
    sw = bufs_sems[3 * NB:4 * NB]
    wid = lax.axis_index("s") * NC + lax.axis_index("c")
    b0 = wid * CH
    pltpu.sync_copy(xt_hbm.at[:, pl.ds(b0, CH)], idx_v)

    @pl.loop(0, SEQ)
    def _prep(s):
      for j in range(CH // L):
        par_v[s, pl.ds(j * L, L)] = lax.shift_right_logical(
            idx_v[s, pl.ds(j * L, L)], 1)

    def out_slice(s):
      return out_hbm.at[s, :, pl.ds(b0, CH)]

    def start_gather(s, b):
      pltpu.async_copy(tp_hbm.at[par_v.at[s]], gbufs[b], sg[b])

    iota = lax.iota(jnp.int32, L)

    for b in range(NB):
      start_gather(b, b)

    @pl.loop(0, SEQ, step=NB)
    def _chunks(s0):
      for b in range(NB):
        s = s0 + b
        pltpu.make_async_copy(tp_hbm.at[par_v.at[s]], gbufs[b], sg[b]).wait()

        @pl.when(s >= NB)
        def _():
          pltpu.make_async_copy(tbufs[b], out_slice(s - NB), sw[b]).wait()

        for j in range(CH // L):
          rows = iota + (j * L)
          off = (idx_v[s, pl.ds(j * L, L)] & 1) * D

          @pl.loop(0, D)
          def _sel(d):
            tbufs[b][d, pl.ds(j * L, L)] = plsc.load_gather(
                gbufs[b], [rows, off + d])

        pltpu.async_copy(tbufs[b], out_slice(s), sw[b])

        @pl.when(s + NB < SEQ)
        def _():
          start_gather(s + NB, b)

    for b in range(NB):
      s = SEQ - NB + b
      pltpu.make_async_copy(tbufs[b], out_slice(s), sw[b]).wait()

  return lookup


_lookup = _make_lookup()


@jax.jit
def kernel(x, table):
  tp = table.reshape(VOCAB // 2, PW)
  o2 = _lookup(tp, x.T.astype(jnp.int32))
  return o2.transpose(2, 0, 1)
